# baseline (speedup 1.0000x reference)
import functools

import jax
import jax.numpy as jnp
from jax import lax
from jax.experimental import pallas as pl
from jax.experimental.pallas import tpu as pltpu
from jax.experimental.pallas import tpu_sc as plsc

_VOCAB = 1000000
_EMBED = 64
_B = 16384
_L = 50
_NC = 2
_NS = 16
_NW = _NC * _NS
_BPW = _B // _NW
_KC = 8
_NBUF = 2
_CHUNKS = _BPW // _KC


def _sc_gather(idx, table):
    mesh = plsc.VectorSubcoreMesh(core_axis_name="c", subcore_axis_name="s")

    @functools.partial(
        pl.kernel,
        out_type=jax.ShapeDtypeStruct((_B, _L, _EMBED), jnp.float32),
        mesh=mesh,
        scratch_types=[
            pltpu.VMEM((_NBUF, _KC, _L), jnp.int32),
            pltpu.VMEM((_NBUF, _KC, _L, _EMBED), jnp.float32),
            pltpu.SemaphoreType.DMA,
            pltpu.SemaphoreType.DMA,
        ],
        compiler_params=pltpu.CompilerParams(use_tc_tiling_on_sc=False),
    )
    def k(idx_hbm, table_hbm, out_hbm, idx_v, rows_v, sem0, sem1):
        sems = (sem0, sem1)
        wid = lax.axis_index("s") * _NC + lax.axis_index("c")
        base_b = wid * _BPW

        def load_and_fire(ci, b):
            r = base_b + ci * _KC
            pltpu.sync_copy(idx_hbm.at[pl.ds(r, _KC)], idx_v.at[b])
            for j in range(_KC):
                pltpu.async_copy(
                    table_hbm.at[idx_v.at[b].at[j]], rows_v.at[b].at[j], sems[b]
                )

        def drain_and_store(ci, b):
            for j in range(_KC):
                pltpu.make_async_copy(
                    table_hbm.at[idx_v.at[b].at[j]], rows_v.at[b].at[j], sems[b]
                ).wait()
            r = base_b + ci * _KC
            pltpu.sync_copy(rows_v.at[b], out_hbm.at[pl.ds(r, _KC)])

        for b in range(_NBUF):
            load_and_fire(b, b)

        @pl.loop(0, _CHUNKS - _NBUF, step=_NBUF)
        def _chunk(i):
            for b in range(_NBUF):
                drain_and_store(i + b, b)
                load_and_fire(i + b + _NBUF, b)

        for b in range(_NBUF):
            drain_and_store(_CHUNKS - _NBUF + b, b)

    return k(idx, table)


def kernel(inputs, embeddings):
    return _sc_gather(inputs.astype(jnp.int32), embeddings)
